# baseline (speedup 1.0000x reference)
import jax
import jax.numpy as jnp
from jax import lax
from jax.experimental import pallas as pl
from jax.experimental.pallas import tpu as pltpu
from jax.experimental.pallas import tpu_sc as plsc

_ROWS = 128
_N = 32768
_K = 64
_L = 16
_NVEC = _N // _L
_NC = 2
_NS = 16
_NW = _NC * _NS
_RPW = _ROWS // _NW
_NFINE = 4096
_RS = 24
_INT_MIN = jnp.int32(-2147483647 - 1)


def _keyify(v):
    u = lax.bitcast_convert_type(v, jnp.int32)
    return jnp.where(u >= 0, u, u ^ jnp.int32(0x7FFFFFFF))


def _popcount(mask):
    r = plsc.all_reduce_population_count(mask)
    return r[0] if r.ndim else r


def _klo_of_bin(p20):
    return jnp.where(p20 == -2048, _INT_MIN, (p20 << 20) - 1)


def _chunk_step(hist, base, acc, kneed, lanes, zeros):
    v = hist[pl.ds(base, _L)]
    hist[pl.ds(base, _L)] = zeros
    cs = jnp.cumsum(jnp.flip(v, 0))
    i0 = _popcount(acc + cs < kneed)
    cs_prev = jnp.sum(jnp.where(lanes == i0 - 1, cs, 0))
    return cs, i0, cs_prev


def _scan_desc(hist, kneed, nchunks):
    lanes = lax.iota(jnp.int32, _L)
    zeros = jnp.zeros((_L,), jnp.int32)

    def step(c, carry):
        acc, found, bstar, kp = carry
        base = (nchunks - 1 - c) * _L
        cs, i0, cs_prev = _chunk_step(hist, base, acc, kneed, lanes, zeros)
        hit = i0 < _L
        upd = jnp.logical_and(hit, found == 0)
        bstar = jnp.where(upd, base + _L - 1 - i0, bstar)
        kp = jnp.where(upd, kneed - acc - cs_prev, kp)
        found = jnp.where(hit, jnp.int32(1), found)
        acc = acc + cs[_L - 1]
        return acc, found, bstar, kp

    init = (jnp.int32(0), jnp.int32(0), jnp.int32(0), jnp.int32(0))
    _, _, bstar, kp = lax.fori_loop(0, nchunks, step, init)
    return bstar, kp


def _scan_fine(hist, kneed, cstart):
    lanes = lax.iota(jnp.int32, _L)
    zeros = jnp.zeros((_L,), jnp.int32)

    def cond(carry):
        return jnp.logical_and(carry[1] == 0, carry[0] >= 0)

    def body(carry):
        c, found, acc, bstar, kp = carry
        base = c * _L
        cs, i0, cs_prev = _chunk_step(hist, base, acc, kneed, lanes, zeros)
        hit = i0 < _L
        bstar = jnp.where(hit, base + _L - 1 - i0, bstar)
        kp = jnp.where(hit, kneed - acc - cs_prev, kp)
        found = jnp.where(hit, jnp.int32(1), found)
        return c - 1, found, acc + cs[_L - 1], bstar, kp

    init = (cstart, jnp.int32(0), jnp.int32(0), jnp.int32(0), jnp.int32(0))
    _, _, _, bstar, kp = lax.while_loop(cond, body, init)
    return bstar, kp


def _sc_body(x_hbm, out_hbm, buf0, buf1, cand, histf, histc, si0, si1, so0, so1):
    lanes = lax.iota(jnp.int32, _L)
    ones = jnp.ones((_L,), jnp.int32)
    zeros = jnp.zeros((_L,), jnp.int32)
    wid = lax.axis_index("s") * _NC + lax.axis_index("c")
    row0 = wid * _RPW

    def clrf(i, c):
        histf[pl.ds(i * _L, _L)] = zeros
        return c
    lax.fori_loop(0, _NFINE // _L, clrf, 0)

    def clrc(i, c):
        histc[pl.ds(i * _L, _L)] = zeros
        return c
    lax.fori_loop(0, 16, clrc, 0)

    def process(buf, row):
        neg_inf = jnp.full((_L,), _INT_MIN)

        @plsc.parallel_loop(0, _NVEC // _L, unroll=4, carry=neg_inf)
        def p_samp(i, mx):
            k = _keyify(buf[pl.ds(i * 256, _L)])
            plsc.addupdate_scatter(histf, [(k >> 20) + 2048], ones)
            return jnp.maximum(mx, k)

        smax = jnp.max(p_samp)
        cstart = ((smax >> 20) + 2048) >> 4
        fbh, _ = _scan_fine(histf, jnp.int32(_RS), cstart)
        klo = _klo_of_bin(fbh - 2048)

        @plsc.parallel_loop(0, _NFINE // _L, unroll=8)
        def p_clr(i):
            histf[pl.ds(i * _L, _L)] = zeros

        @plsc.parallel_loop(0, _NVEC, unroll=4, carry=jnp.int32(0))
        def p_main(i, off):
            v = buf[pl.ds(i * _L, _L)]
            k = _keyify(v)
            msk = k > klo
            buf[pl.ds(i * _L, _L)] = jnp.where(msk, v, jnp.float32(0))
            plsc.store_compressed(cand.at[pl.ds(off, _L)], i * _L + lanes, mask=msk)
            return off + _popcount(msk)

        m = p_main

        need_fb = m < _K

        @pl.when(need_fb)
        def _():
            pltpu.sync_copy(x_hbm.at[row], buf)

        nfb = jnp.where(need_fb, _NVEC, 0)

        def fb_hist(i, c):
            k = _keyify(buf[pl.ds(i * _L, _L)])
            plsc.addupdate_scatter(histf, [(k >> 20) + 2048], ones)
            return c

        lax.fori_loop(0, nfb, fb_hist, 0)
        fb2, _ = _scan_fine(histf, jnp.int32(_K),
                            jnp.where(need_fb, jnp.int32(_NFINE // _L - 1),
                                      jnp.int32(-1)))
        klo2 = _klo_of_bin(fb2 - 2048)
        nclr = jnp.where(need_fb, _NFINE // _L, 0)
        lax.fori_loop(0, nclr, clrf, 0)

        def fb_main(i, off):
            v = buf[pl.ds(i * _L, _L)]
            k = _keyify(v)
            msk = k > klo2
            buf[pl.ds(i * _L, _L)] = jnp.where(msk, v, jnp.float32(0))
            plsc.store_compressed(cand.at[pl.ds(off, _L)], i * _L + lanes, mask=msk)
            return off + _popcount(msk)

        m2 = lax.fori_loop(0, nfb, fb_main, jnp.int32(0))
        m = jnp.where(need_fb, m2, m)

        nv = (m + _L - 1) // _L

        def p_byte(i, c):
            pos = cand[pl.ds(i * _L, _L)]
            valid = (i * _L + lanes) < m
            k = _keyify(plsc.load_gather(buf, [pos], mask=valid))
            plsc.addupdate_scatter(histc, [(k >> 24) + 128], ones, mask=valid)
            return c

        lax.fori_loop(0, nv, p_byte, 0)
        b3, kneed = _scan_desc(histc, jnp.int32(_K), 16)
        prefix = b3 - 128

        def level(prefix, pshift, shift, kneed):
            def ph(i, c):
                pos = cand[pl.ds(i * _L, _L)]
                valid = (i * _L + lanes) < m
                k = _keyify(plsc.load_gather(buf, [pos], mask=valid))
                ok = jnp.logical_and(valid, (k >> pshift) == prefix)
                plsc.addupdate_scatter(histc, [(k >> shift) & 255], ones, mask=ok)
                return c

            lax.fori_loop(0, nv, ph, 0)
            bs, kneed2 = _scan_desc(histc, kneed, 16)
            return bs, kneed2

        b2, kneed = level(prefix, 24, 16, kneed)
        prefix = prefix * 256 + b2
        b1, kneed = level(prefix, 16, 8, kneed)
        prefix = prefix * 256 + b1
        b0, kneed = level(prefix, 8, 0, kneed)
        t = prefix * 256 + b0
        mfin = kneed
        tmax = jnp.maximum(t, jnp.int32(0))

        def p_fix(i, eq_seen):
            pos = cand[pl.ds(i * _L, _L)]
            valid = (i * _L + lanes) < m
            v = plsc.load_gather(buf, [pos], mask=valid)
            k = _keyify(v)
            eq = jnp.logical_and(k == t, valid)
            eqc = jnp.cumsum(eq.astype(jnp.int32))
            sel = jnp.logical_or(k > tmax,
                                 jnp.logical_and(eq, eq_seen + eqc <= mfin))
            outv = jnp.where(jnp.logical_and(sel, k > 0), v, jnp.float32(0))
            plsc.store_scatter(buf, [pos], outv, mask=valid)
            return eq_seen + plsc.all_reduce_population_count(eq)

        lax.fori_loop(0, nv, p_fix, jnp.zeros((_L,), jnp.int32))

    bufs = (buf0, buf1)
    sin = (si0, si1)
    sout = (so0, so1)
    in_h = [None] * _RPW
    out_h = [None] * _RPW
    in_h[0] = pltpu.async_copy(x_hbm.at[row0], buf0, si0)
    for r in range(_RPW):
        b = bufs[r % 2]
        if r + 1 < _RPW:
            if r >= 1:
                out_h[r - 1].wait()
            in_h[r + 1] = pltpu.async_copy(
                x_hbm.at[row0 + r + 1], bufs[(r + 1) % 2], sin[(r + 1) % 2])
        in_h[r].wait()
        process(b, row0 + r)
        out_h[r] = pltpu.async_copy(b, out_hbm.at[row0 + r], sout[r % 2])
    out_h[_RPW - 2].wait()
    out_h[_RPW - 1].wait()


@jax.jit
def kernel(x):
    mesh = plsc.VectorSubcoreMesh(core_axis_name="c", subcore_axis_name="s")
    run = pl.kernel(
        _sc_body,
        out_type=jax.ShapeDtypeStruct((_ROWS, _N), jnp.float32),
        mesh=mesh,
        scratch_types=[
            pltpu.VMEM((_N,), jnp.float32),
            pltpu.VMEM((_N,), jnp.float32),
            pltpu.VMEM((_N + _L,), jnp.int32),
            pltpu.VMEM((_NFINE,), jnp.int32),
            pltpu.VMEM((256,), jnp.int32),
            pltpu.SemaphoreType.DMA,
            pltpu.SemaphoreType.DMA,
            pltpu.SemaphoreType.DMA,
            pltpu.SemaphoreType.DMA,
        ],
        compiler_params=pltpu.CompilerParams(needs_layout_passes=False),
    )
    return run(x)
